# baseline (speedup 1.0000x reference)
import jax
import jax.numpy as jnp
from jax import lax
from jax.experimental import pallas as pl
from jax.experimental.pallas import tpu as pltpu
from jax.experimental.pallas import tpu_sc as plsc

D = 64
SCALE = 8.0
W = 128
NC, NS = 2, 16
NW = NC * NS
K = 3


def kernel(x, table):
    B, S = x.shape
    N = B * S
    V = table.shape[0]
    npt = N // NW
    cpt = npt // W
    idx = x.reshape(1, N)
    t2 = table.reshape(V // 2, 2 * D)
    mesh = plsc.VectorSubcoreMesh(core_axis_name="c", subcore_axis_name="s")

    @pl.kernel(
        out_type=jax.ShapeDtypeStruct((N, D), jnp.float32),
        mesh=mesh,
        scratch_types=[
            pltpu.VMEM((npt,), jnp.int32),
            pltpu.VMEM((K, W), jnp.int32),
            pltpu.VMEM((K, W, 2 * D), jnp.float32),
            pltpu.VMEM((K, W, D), jnp.float32),
            pltpu.SemaphoreType.DMA,
            pltpu.SemaphoreType.DMA((K,)),
            pltpu.SemaphoreType.DMA((K,)),
        ],
        compiler_params=pltpu.CompilerParams(use_tc_tiling_on_sc=True),
    )
    def k(t2_hbm, i_hbm, o_hbm, idx_v, pidx_v, gbuf, wbuf, isem, gsem, osem):
        wid = lax.axis_index("c") * NS + lax.axis_index("s")

        pltpu.async_copy(
            i_hbm.at[0, pl.ds(pl.multiple_of(wid * npt, 128), npt)],
            idx_v,
            isem,
        ).wait()

        def issue_gather(g, b):
            @pl.loop(0, W, step=16)
            def _(r):
                pidx_v.at[b, pl.ds(r, 16)][...] = (
                    idx_v.at[pl.ds(g * W + r, 16)][...] >> 1
                )

            pltpu.async_copy(t2_hbm.at[pidx_v.at[b]], gbuf.at[b], gsem.at[b])

        for b in range(K):
            issue_gather(b, b)

        @pl.loop(0, cpt, step=K)
        def _(g0):
            for b in range(K):
                g = g0 + b
                pltpu.make_async_copy(
                    t2_hbm.at[pl.ds(0, W)], gbuf.at[b], gsem.at[b]
                ).wait()

                @pl.when(g0 >= K)
                def _():
                    pltpu.make_async_copy(
                        wbuf.at[b], o_hbm.at[pl.ds(0, W)], osem.at[b]
                    ).wait()

                @pl.loop(0, W, step=16)
                def _(r):
                    iv = idx_v[pl.ds(g * W + r, 16)]
                    for rr in range(16):
                        off = (iv[rr] & 1) * D
                        for c in range(0, D, 16):
                            wbuf.at[b, r + rr, pl.ds(c, 16)][...] = (
                                gbuf.at[b, r + rr, pl.ds(off + c, 16)][...]
                                * SCALE
                            )

                @pl.when(g0 + K < cpt)
                def _():
                    issue_gather(g + K, b)

                pltpu.async_copy(
                    wbuf.at[b],
                    o_hbm.at[
                        pl.ds(pl.multiple_of(wid * npt + g * W, 128), W)
                    ],
                    osem.at[b],
                )

        for b in range(K):
            pltpu.make_async_copy(
                wbuf.at[b], o_hbm.at[pl.ds(0, W)], osem.at[b]
            ).wait()

    out = k(t2, idx)
    return out.reshape(B, S, D)


# speedup vs baseline: 1.1546x; 1.1546x over previous
import jax
import jax.numpy as jnp
from jax import lax
from jax.experimental import pallas as pl
from jax.experimental.pallas import tpu as pltpu
from jax.experimental.pallas import tpu_sc as plsc

D = 64
DP = 128
SCALE = 8.0
W = 128
NC, NS = 2, 16
NW = NC * NS
K = 3


def kernel(x, table):
    B, S = x.shape
    N = B * S
    V = table.shape[0]
    npt = N // NW
    cpt = npt // W
    idx = x.reshape(1, N)
    t3 = jnp.pad(table, ((0, 0), (0, DP - D)))
    mesh = plsc.VectorSubcoreMesh(core_axis_name="c", subcore_axis_name="s")

    @pl.kernel(
        out_type=jax.ShapeDtypeStruct((N, D), jnp.float32),
        mesh=mesh,
        scratch_types=[
            pltpu.VMEM((npt,), jnp.int32),
            pltpu.VMEM((K, W), jnp.int32),
            pltpu.VMEM((K, W, DP), jnp.float32),
            pltpu.VMEM((K, W, D), jnp.float32),
            pltpu.SemaphoreType.DMA,
            pltpu.SemaphoreType.DMA((K,)),
            pltpu.SemaphoreType.DMA((K,)),
        ],
        compiler_params=pltpu.CompilerParams(use_tc_tiling_on_sc=True),
    )
    def k(t3_hbm, i_hbm, o_hbm, idx_v, cidx_v, gbuf, wbuf, isem, gsem, osem):
        wid = lax.axis_index("c") * NS + lax.axis_index("s")

        pltpu.async_copy(
            i_hbm.at[0, pl.ds(pl.multiple_of(wid * npt, 128), npt)],
            idx_v,
            isem,
        ).wait()

        def issue_gather(g, b):
            @pl.loop(0, W, step=16)
            def _(r):
                cidx_v.at[b, pl.ds(r, 16)][...] = idx_v.at[
                    pl.ds(g * W + r, 16)
                ][...]

            pltpu.async_copy(t3_hbm.at[cidx_v.at[b]], gbuf.at[b], gsem.at[b])

        for b in range(K):
            issue_gather(b, b)

        @pl.loop(0, cpt, step=K)
        def _(g0):
            for b in range(K):
                g = g0 + b
                pltpu.make_async_copy(
                    t3_hbm.at[pl.ds(0, W)], gbuf.at[b], gsem.at[b]
                ).wait()

                @pl.when(g0 >= K)
                def _():
                    pltpu.make_async_copy(
                        wbuf.at[b], o_hbm.at[pl.ds(0, W)], osem.at[b]
                    ).wait()

                @pl.loop(0, W, step=4)
                def _(r):
                    for rr in range(4):
                        for c in range(0, D, 16):
                            wbuf.at[b, r + rr, pl.ds(c, 16)][...] = (
                                gbuf.at[b, r + rr, pl.ds(c, 16)][...] * SCALE
                            )

                @pl.when(g0 + K < cpt)
                def _():
                    issue_gather(g + K, b)

                pltpu.async_copy(
                    wbuf.at[b],
                    o_hbm.at[
                        pl.ds(pl.multiple_of(wid * npt + g * W, 128), W)
                    ],
                    osem.at[b],
                )

        for b in range(K):
            pltpu.make_async_copy(
                wbuf.at[b], o_hbm.at[pl.ds(0, W)], osem.at[b]
            ).wait()

    out = k(t3, idx)
    return out.reshape(B, S, D)
